# Initial kernel scaffold; baseline (speedup 1.0000x reference)
import jax
import jax.numpy as jnp
from jax.experimental import pallas as pl


def kernel(x, edge_index, edge_weight, W1, W2):
    raise NotImplementedError("write your pallas kernel here")


# speedup vs baseline: 5.7328x; 5.7328x over previous
import functools

import jax
import jax.numpy as jnp
from jax import lax
from jax.experimental import pallas as pl
from jax.experimental.pallas import tpu as pltpu
from jax.experimental.pallas import tpu_sc as plsc

_NC = 2
_NS = 16
_NW = _NC * _NS
_CH = 128


def _make_edge_agg(n_nodes: int, d: int, n_chunks: int):
    zc = 80
    n_zc = n_nodes // zc
    assert n_nodes % zc == 0
    mesh = plsc.VectorSubcoreMesh(core_axis_name="c", subcore_axis_name="s")

    @functools.partial(
        pl.kernel,
        out_type=jax.ShapeDtypeStruct((_NC, n_nodes, d), jnp.float32),
        mesh=mesh,
        scratch_types=[
            pltpu.VMEM((_CH,), jnp.int32),
            pltpu.VMEM((_CH,), jnp.int32),
            pltpu.VMEM((_CH,), jnp.float32),
            pltpu.VMEM((_CH, d), jnp.float32),
            pltpu.VMEM_SHARED((n_nodes, d), jnp.float32),
            pltpu.SemaphoreType.DMA,
        ],
        compiler_params=pltpu.CompilerParams(
            use_tc_tiling_on_sc=(d % 128 == 0)),
    )
    def agg(feat_hbm, src_hbm, dst_hbm, ew_hbm, out_hbm,
            sidx, didx, wts, rows, acc, sem):
        cid = lax.axis_index("c")
        sid = lax.axis_index("s")
        wid = sid * _NC + cid

        def _zrow(i, _):
            for j in range(d // 16):
                rows[i, pl.ds(j * 16, 16)] = jnp.zeros((16,), jnp.float32)
            return 0
        lax.fori_loop(0, zc, _zrow, 0)

        def _zcopy(m, _):
            z = sid + m * _NS
            pltpu.sync_copy(rows.at[pl.ds(0, zc)], acc.at[pl.ds(z * zc, zc)])
            return 0
        lax.fori_loop(0, (n_zc - sid + _NS - 1) // _NS, _zcopy, 0)
        plsc.subcore_barrier()

        nk = (n_chunks - wid + _NW - 1) // _NW

        def body(k, _):
            c = wid + k * _NW
            pltpu.sync_copy(src_hbm.at[c], sidx)
            pltpu.sync_copy(dst_hbm.at[c], didx)
            pltpu.sync_copy(ew_hbm.at[c], wts)
            pltpu.async_copy(feat_hbm.at[sidx], rows, sem).wait()

            def scale(k, _):
                w16 = wts[pl.ds(k * 16, 16)]
                for l in range(16):
                    w = w16[l]
                    i = k * 16 + l
                    for j in range(d // 16):
                        sl = pl.ds(j * 16, 16)
                        rows[i, sl] = rows[i, sl] * w
                return 0
            lax.fori_loop(0, _CH // 16, scale, 0)

            pltpu.sync_copy(rows, acc.at[didx], add=True)
            return 0
        lax.fori_loop(0, nk, body, 0)

        plsc.subcore_barrier()

        def _ocopy(m, _):
            z = sid + m * _NS
            pltpu.sync_copy(acc.at[pl.ds(z * zc, zc)],
                            out_hbm.at[cid, pl.ds(z * zc, zc)])
            return 0
        lax.fori_loop(0, (n_zc - sid + _NS - 1) // _NS, _ocopy, 0)

    return agg


def _linear2(P, W1, W2, n_nodes: int):
    R = 2000
    assert n_nodes % R == 0
    d = P.shape[2]
    n_cls = W2.shape[0]

    def mm(p_ref, w1_ref, w2_ref, g_ref):
        a = p_ref[0] + p_ref[1]
        h = lax.dot_general(a, w1_ref[...], (((1,), (1,)), ((), ())),
                            preferred_element_type=jnp.float32)
        h = jnp.maximum(h, 0.0)
        g_ref[...] = lax.dot_general(h, w2_ref[...], (((1,), (1,)), ((), ())),
                                     preferred_element_type=jnp.float32)

    return pl.pallas_call(
        mm,
        grid=(n_nodes // R,),
        in_specs=[
            pl.BlockSpec((2, R, d), lambda i: (0, i, 0)),
            pl.BlockSpec((d, d), lambda i: (0, 0)),
            pl.BlockSpec((n_cls, d), lambda i: (0, 0)),
        ],
        out_specs=pl.BlockSpec((R, n_cls), lambda i: (i, 0)),
        out_shape=jax.ShapeDtypeStruct((n_nodes, n_cls), jnp.float32),
    )(P, W1, W2)


def _log_softmax_sum(Q, n_nodes: int):
    R = 2000
    n_cls = Q.shape[2]

    def lsm(q_ref, o_ref):
        s = q_ref[0] + q_ref[1]
        m = jnp.max(s, axis=-1, keepdims=True)
        e = jnp.exp(s - m)
        lse = jnp.log(jnp.sum(e, axis=-1, keepdims=True))
        o_ref[...] = s - m - lse

    return pl.pallas_call(
        lsm,
        grid=(n_nodes // R,),
        in_specs=[pl.BlockSpec((2, R, n_cls), lambda i: (0, i, 0))],
        out_specs=pl.BlockSpec((R, n_cls), lambda i: (i, 0)),
        out_shape=jax.ShapeDtypeStruct((n_nodes, n_cls), jnp.float32),
    )(Q)


def kernel(x, edge_index, edge_weight, W1, W2):
    n_nodes, d_feat = x.shape
    n_edges = edge_weight.shape[0]
    assert n_edges % _CH == 0
    n_chunks = n_edges // _CH

    src2d = edge_index[0].astype(jnp.int32).reshape(n_chunks, _CH)
    dst2d = edge_index[1].astype(jnp.int32).reshape(n_chunks, _CH)
    ew2d = edge_weight.reshape(n_chunks, _CH)

    agg_x = _make_edge_agg(n_nodes, d_feat, n_chunks)
    P = agg_x(x, src2d, dst2d, ew2d)
    g = _linear2(P, W1, W2, n_nodes)
    agg_g = _make_edge_agg(n_nodes, W2.shape[0], n_chunks)
    Q = agg_g(g, src2d, dst2d, ew2d)
    return _log_softmax_sum(Q, n_nodes)


# speedup vs baseline: 5.8814x; 1.0259x over previous
import functools

import jax
import jax.numpy as jnp
from jax import lax
from jax.experimental import pallas as pl
from jax.experimental.pallas import tpu as pltpu
from jax.experimental.pallas import tpu_sc as plsc

_NC = 2
_NS = 16
_NW = _NC * _NS
_CH = 128


_G = 16


def _make_edge_agg(n_nodes: int, d: int, n_chunks: int):
    assert n_chunks % (_NW * _G) == 0
    K = n_chunks // _NW
    NG = K // _G
    zc = 80
    n_zc = n_nodes // zc
    assert n_nodes % zc == 0
    mesh = plsc.VectorSubcoreMesh(core_axis_name="c", subcore_axis_name="s")

    @functools.partial(
        pl.kernel,
        out_type=jax.ShapeDtypeStruct((_NC, n_nodes, d), jnp.float32),
        mesh=mesh,
        scratch_types=[
            [pltpu.VMEM((_G, _CH), jnp.int32) for _ in range(2)],
            [pltpu.VMEM((_G, _CH), jnp.int32) for _ in range(2)],
            [pltpu.VMEM((_G, _CH), jnp.float32) for _ in range(2)],
            [pltpu.VMEM((_CH, d), jnp.float32) for _ in range(2)],
            pltpu.VMEM_SHARED((n_nodes, d), jnp.float32),
            [pltpu.SemaphoreType.DMA for _ in range(2)],
            [pltpu.SemaphoreType.DMA for _ in range(2)],
            [pltpu.SemaphoreType.DMA for _ in range(2)],
        ],
        compiler_params=pltpu.CompilerParams(
            use_tc_tiling_on_sc=(d % 128 == 0)),
    )
    def agg(feat_hbm, src_hbm, dst_hbm, ew_hbm, out_hbm,
            sblk, dblk, wblk, rows, acc, semi, gsem, ssem):
        cid = lax.axis_index("c")
        sid = lax.axis_index("s")
        wid = sid * _NC + cid
        base = wid * K

        def _fetch_idx(gg, gs):
            pltpu.async_copy(src_hbm.at[pl.ds(base + gg * _G, _G)],
                             sblk[gs], semi[gs])
            pltpu.async_copy(dst_hbm.at[pl.ds(base + gg * _G, _G)],
                             dblk[gs], semi[gs])
            pltpu.async_copy(ew_hbm.at[pl.ds(base + gg * _G, _G)],
                             wblk[gs], semi[gs])

        def _wait_idx(gs):
            pltpu.make_async_copy(src_hbm.at[pl.ds(base, _G)], sblk[gs],
                                  semi[gs]).wait()
            pltpu.make_async_copy(dst_hbm.at[pl.ds(base, _G)], dblk[gs],
                                  semi[gs]).wait()
            pltpu.make_async_copy(ew_hbm.at[pl.ds(base, _G)], wblk[gs],
                                  semi[gs]).wait()

        def _gather(gs, t, j):
            pltpu.async_copy(feat_hbm.at[sblk[gs].at[t]], rows[j], gsem[j])

        def _gather_wait(gs, j):
            pltpu.make_async_copy(feat_hbm.at[sblk[gs].at[0]], rows[j],
                                  gsem[j]).wait()

        def _scatter(gs, t, j):
            pltpu.async_copy(rows[j], acc.at[dblk[gs].at[t]], ssem[j],
                             add=True)

        def _scatter_wait(gs, j):
            pltpu.make_async_copy(rows[j], acc.at[dblk[gs].at[0]],
                                  ssem[j]).wait()

        def _scale(gs, t, j):
            def srow(m, _):
                w16 = wblk[gs][t, pl.ds(m * 16, 16)]
                for l in range(16):
                    w = w16[l]
                    i = m * 16 + l
                    for q in range(d // 16):
                        sl = pl.ds(q * 16, 16)
                        rows[j][i, sl] = rows[j][i, sl] * w
                return 0
            lax.fori_loop(0, _CH // 16, srow, 0)

        _fetch_idx(0, 0)
        if NG > 1:
            _fetch_idx(1, 1)

        def _zrow(i, _):
            for j in range(d // 16):
                rows[0][i, pl.ds(j * 16, 16)] = jnp.zeros((16,), jnp.float32)
            return 0
        lax.fori_loop(0, zc, _zrow, 0)

        def _zcopy(m, _):
            z = sid + m * _NS
            pltpu.sync_copy(rows[0].at[pl.ds(0, zc)], acc.at[pl.ds(z * zc, zc)])
            return 0
        lax.fori_loop(0, (n_zc - sid + _NS - 1) // _NS, _zcopy, 0)

        _wait_idx(0)
        _gather(0, 0, 0)
        _gather(0, 1, 1)
        plsc.subcore_barrier()

        for gg in range(NG):
            gs = gg % 2

            def inner(it, _):
                for j in range(2):
                    t = 2 * it + j
                    _gather_wait(gs, j)
                    _scale(gs, t, j)
                    _scatter(gs, t, j)
                for j in range(2):
                    t = 2 * it + j
                    _scatter_wait(gs, j)
                    _gather(gs, t + 2, j)
                return 0
            lax.fori_loop(0, (_G - 2) // 2, inner, 0)

            for j, t in ((0, _G - 2), (1, _G - 1)):
                _gather_wait(gs, j)
                _scale(gs, t, j)
                _scatter(gs, t, j)
            if gg < NG - 1:
                _wait_idx(1 - gs)
            for j in range(2):
                _scatter_wait(gs, j)
                if gg < NG - 1:
                    _gather(1 - gs, j, j)
            if gg < NG - 2:
                _fetch_idx(gg + 2, gs)

        plsc.subcore_barrier()

        def _ocopy(m, _):
            z = sid + m * _NS
            pltpu.sync_copy(acc.at[pl.ds(z * zc, zc)],
                            out_hbm.at[cid, pl.ds(z * zc, zc)])
            return 0
        lax.fori_loop(0, (n_zc - sid + _NS - 1) // _NS, _ocopy, 0)

    return agg


def _linear2(P, W1, W2, n_nodes: int):
    R = 2000
    assert n_nodes % R == 0
    d = P.shape[2]
    n_cls = W2.shape[0]

    def mm(p_ref, w1_ref, w2_ref, g_ref):
        a = p_ref[0] + p_ref[1]
        h = lax.dot_general(a, w1_ref[...], (((1,), (1,)), ((), ())),
                            preferred_element_type=jnp.float32)
        h = jnp.maximum(h, 0.0)
        g_ref[...] = lax.dot_general(h, w2_ref[...], (((1,), (1,)), ((), ())),
                                     preferred_element_type=jnp.float32)

    return pl.pallas_call(
        mm,
        grid=(n_nodes // R,),
        in_specs=[
            pl.BlockSpec((2, R, d), lambda i: (0, i, 0)),
            pl.BlockSpec((d, d), lambda i: (0, 0)),
            pl.BlockSpec((n_cls, d), lambda i: (0, 0)),
        ],
        out_specs=pl.BlockSpec((R, n_cls), lambda i: (i, 0)),
        out_shape=jax.ShapeDtypeStruct((n_nodes, n_cls), jnp.float32),
    )(P, W1, W2)


def _log_softmax_sum(Q, n_nodes: int):
    R = 2000
    n_cls = Q.shape[2]

    def lsm(q_ref, o_ref):
        s = q_ref[0] + q_ref[1]
        m = jnp.max(s, axis=-1, keepdims=True)
        e = jnp.exp(s - m)
        lse = jnp.log(jnp.sum(e, axis=-1, keepdims=True))
        o_ref[...] = s - m - lse

    return pl.pallas_call(
        lsm,
        grid=(n_nodes // R,),
        in_specs=[pl.BlockSpec((2, R, n_cls), lambda i: (0, i, 0))],
        out_specs=pl.BlockSpec((R, n_cls), lambda i: (i, 0)),
        out_shape=jax.ShapeDtypeStruct((n_nodes, n_cls), jnp.float32),
    )(Q)


def kernel(x, edge_index, edge_weight, W1, W2):
    n_nodes, d_feat = x.shape
    n_edges = edge_weight.shape[0]
    grp = _CH * _NW * _G
    n_pad = -(-n_edges // grp) * grp - n_edges
    n_chunks = (n_edges + n_pad) // _CH

    pad_i = jnp.zeros((n_pad,), jnp.int32)
    src2d = jnp.concatenate(
        [edge_index[0].astype(jnp.int32), pad_i]).reshape(n_chunks, _CH)
    dst2d = jnp.concatenate(
        [edge_index[1].astype(jnp.int32), pad_i]).reshape(n_chunks, _CH)
    ew2d = jnp.concatenate(
        [edge_weight, jnp.zeros((n_pad,), jnp.float32)]).reshape(n_chunks, _CH)

    agg_x = _make_edge_agg(n_nodes, d_feat, n_chunks)
    P = agg_x(x, src2d, dst2d, ew2d)
    g = _linear2(P, W1, W2, n_nodes)
    agg_g = _make_edge_agg(n_nodes, W2.shape[0], n_chunks)
    Q = agg_g(g, src2d, dst2d, ew2d)
    return _log_softmax_sum(Q, n_nodes)


# speedup vs baseline: 12.4865x; 2.1231x over previous
import functools

import jax
import jax.numpy as jnp
from jax import lax
from jax.experimental import pallas as pl
from jax.experimental.pallas import tpu as pltpu
from jax.experimental.pallas import tpu_sc as plsc

_NC = 2
_NS = 16
_NW = _NC * _NS
_CH = 128


_G = 16


def _make_edge_agg(n_nodes: int, d: int, n_chunks: int):
    assert n_chunks % (_NW * _G) == 0
    K = n_chunks // _NW
    NG = K // _G
    zc = 80
    n_zc = n_nodes // zc
    assert n_nodes % zc == 0
    mesh = plsc.VectorSubcoreMesh(core_axis_name="c", subcore_axis_name="s")

    @functools.partial(
        pl.kernel,
        out_type=jax.ShapeDtypeStruct((_NC, n_nodes, d), jnp.float32),
        mesh=mesh,
        scratch_types=[
            [pltpu.VMEM((_G, _CH), jnp.int32) for _ in range(2)],
            [pltpu.VMEM((_G, _CH), jnp.int32) for _ in range(2)],
            [pltpu.VMEM((_G, _CH), jnp.float32) for _ in range(2)],
            [pltpu.VMEM((_CH, d), jnp.float32) for _ in range(2)],
            pltpu.VMEM_SHARED((n_nodes, d), jnp.float32),
            [pltpu.SemaphoreType.DMA for _ in range(2)],
            [pltpu.SemaphoreType.DMA for _ in range(2)],
            [pltpu.SemaphoreType.DMA for _ in range(2)],
        ],
        compiler_params=pltpu.CompilerParams(
            use_tc_tiling_on_sc=(d % 128 == 0)),
    )
    def agg(feat_hbm, src_hbm, dst_hbm, ew_hbm, out_hbm,
            sblk, dblk, wblk, rows, acc, semi, gsem, ssem):
        cid = lax.axis_index("c")
        sid = lax.axis_index("s")
        wid = sid * _NC + cid
        base = wid * K

        def _fetch_idx(gg, gs):
            pltpu.async_copy(src_hbm.at[pl.ds(base + gg * _G, _G)],
                             sblk[gs], semi[gs])
            pltpu.async_copy(dst_hbm.at[pl.ds(base + gg * _G, _G)],
                             dblk[gs], semi[gs])
            pltpu.async_copy(ew_hbm.at[pl.ds(base + gg * _G, _G)],
                             wblk[gs], semi[gs])

        def _wait_idx(gs):
            pltpu.make_async_copy(src_hbm.at[pl.ds(base, _G)], sblk[gs],
                                  semi[gs]).wait()
            pltpu.make_async_copy(dst_hbm.at[pl.ds(base, _G)], dblk[gs],
                                  semi[gs]).wait()
            pltpu.make_async_copy(ew_hbm.at[pl.ds(base, _G)], wblk[gs],
                                  semi[gs]).wait()

        def _gather(gs, t, j):
            pltpu.async_copy(feat_hbm.at[sblk[gs].at[t]], rows[j], gsem[j])

        def _gather_wait(gs, j):
            pltpu.make_async_copy(feat_hbm.at[sblk[gs].at[0]], rows[j],
                                  gsem[j]).wait()

        def _scatter(gs, t, j):
            pltpu.async_copy(rows[j], acc.at[dblk[gs].at[t]], ssem[j],
                             add=True)

        def _scatter_wait(gs, j):
            pltpu.make_async_copy(rows[j], acc.at[dblk[gs].at[0]],
                                  ssem[j]).wait()

        def _scale(gs, t, j):
            def srow(m, _):
                w16 = wblk[gs][t, pl.ds(m * 16, 16)]
                for l in range(16):
                    w = w16[l]
                    i = m * 16 + l
                    for q in range(d // 16):
                        sl = pl.ds(q * 16, 16)
                        rows[j][i, sl] = rows[j][i, sl] * w
                return 0
            lax.fori_loop(0, _CH // 16, srow, 0)

        _fetch_idx(0, 0)
        if NG > 1:
            _fetch_idx(1, 1)

        def _zrow(i, _):
            for j in range(d // 16):
                rows[0][i, pl.ds(j * 16, 16)] = jnp.zeros((16,), jnp.float32)
            return 0
        lax.fori_loop(0, zc, _zrow, 0)

        def _zcopy(m, _):
            z = sid + m * _NS
            pltpu.sync_copy(rows[0].at[pl.ds(0, zc)], acc.at[pl.ds(z * zc, zc)])
            return 0
        lax.fori_loop(0, (n_zc - sid + _NS - 1) // _NS, _zcopy, 0)

        _wait_idx(0)
        _gather(0, 0, 0)
        _gather(0, 1, 1)
        plsc.subcore_barrier()

        for gg in range(NG):
            gs = gg % 2

            def inner(it, _):
                for j in range(2):
                    t = 2 * it + j
                    _gather_wait(gs, j)
                    _scale(gs, t, j)
                    _scatter(gs, t, j)
                for j in range(2):
                    t = 2 * it + j
                    _scatter_wait(gs, j)
                    _gather(gs, t + 2, j)
                return 0
            lax.fori_loop(0, (_G - 2) // 2, inner, 0)

            for j, t in ((0, _G - 2), (1, _G - 1)):
                _gather_wait(gs, j)
                _scale(gs, t, j)
                _scatter(gs, t, j)
            if gg < NG - 1:
                _wait_idx(1 - gs)
            for j in range(2):
                _scatter_wait(gs, j)
                if gg < NG - 1:
                    _gather(1 - gs, j, j)
            if gg < NG - 2:
                _fetch_idx(gg + 2, gs)

        plsc.subcore_barrier()

        def _ocopy(m, _):
            z = sid + m * _NS
            pltpu.sync_copy(acc.at[pl.ds(z * zc, zc)],
                            out_hbm.at[cid, pl.ds(z * zc, zc)])
            return 0
        lax.fori_loop(0, (n_zc - sid + _NS - 1) // _NS, _ocopy, 0)

    return agg


def _linear2(P, W1, W2, n_nodes: int):
    R = 2000
    assert n_nodes % R == 0
    d = P.shape[2]
    n_cls = W2.shape[0]

    def mm(p_ref, w1_ref, w2_ref, g_ref):
        a = p_ref[0] + p_ref[1]
        h = lax.dot_general(a, w1_ref[...], (((1,), (1,)), ((), ())),
                            preferred_element_type=jnp.float32)
        h = jnp.maximum(h, 0.0)
        g_ref[...] = lax.dot_general(h, w2_ref[...], (((1,), (1,)), ((), ())),
                                     preferred_element_type=jnp.float32)

    return pl.pallas_call(
        mm,
        grid=(n_nodes // R,),
        in_specs=[
            pl.BlockSpec((2, R, d), lambda i: (0, i, 0)),
            pl.BlockSpec((d, d), lambda i: (0, 0)),
            pl.BlockSpec((n_cls, d), lambda i: (0, 0)),
        ],
        out_specs=pl.BlockSpec((R, n_cls), lambda i: (i, 0)),
        out_shape=jax.ShapeDtypeStruct((n_nodes, n_cls), jnp.float32),
    )(P, W1, W2)


def _log_softmax_sum(Q, n_nodes: int):
    R = 2000
    n_cls = Q.shape[2]

    def lsm(q_ref, o_ref):
        s = q_ref[0] + q_ref[1]
        m = jnp.max(s, axis=-1, keepdims=True)
        e = jnp.exp(s - m)
        lse = jnp.log(jnp.sum(e, axis=-1, keepdims=True))
        o_ref[...] = s - m - lse

    return pl.pallas_call(
        lsm,
        grid=(n_nodes // R,),
        in_specs=[pl.BlockSpec((2, R, n_cls), lambda i: (0, i, 0))],
        out_specs=pl.BlockSpec((R, n_cls), lambda i: (i, 0)),
        out_shape=jax.ShapeDtypeStruct((n_nodes, n_cls), jnp.float32),
    )(Q)


def kernel(x, edge_index, edge_weight, W1, W2):
    n_nodes, d_feat = x.shape
    n_edges = edge_weight.shape[0]
    grp = _CH * _NW * _G
    n_pad = -(-n_edges // grp) * grp - n_edges
    n_chunks = (n_edges + n_pad) // _CH

    pad_i = jnp.arange(n_pad, dtype=jnp.int32) % n_nodes
    src2d = jnp.concatenate(
        [edge_index[0].astype(jnp.int32), pad_i]).reshape(n_chunks, _CH)
    dst2d = jnp.concatenate(
        [edge_index[1].astype(jnp.int32), pad_i]).reshape(n_chunks, _CH)
    ew2d = jnp.concatenate(
        [edge_weight, jnp.zeros((n_pad,), jnp.float32)]).reshape(n_chunks, _CH)

    agg_x = _make_edge_agg(n_nodes, d_feat, n_chunks)
    P = agg_x(x, src2d, dst2d, ew2d)
    g = _linear2(P, W1, W2, n_nodes)
    agg_g = _make_edge_agg(n_nodes, W2.shape[0], n_chunks)
    Q = agg_g(g, src2d, dst2d, ew2d)
    return _log_softmax_sum(Q, n_nodes)


# speedup vs baseline: 13.6644x; 1.0943x over previous
import functools

import jax
import jax.numpy as jnp
from jax import lax
from jax.experimental import pallas as pl
from jax.experimental.pallas import tpu as pltpu
from jax.experimental.pallas import tpu_sc as plsc

_NC = 2
_NS = 16
_NW = _NC * _NS
_CH = 128


_G = 16


def _make_edge_agg(n_nodes: int, d: int, n_chunks: int):
    assert n_chunks % (_NW * _G) == 0
    K = n_chunks // _NW
    NG = K // _G
    zc = 80
    n_zc = n_nodes // zc
    assert n_nodes % zc == 0
    mesh = plsc.VectorSubcoreMesh(core_axis_name="c", subcore_axis_name="s")

    @functools.partial(
        pl.kernel,
        out_type=jax.ShapeDtypeStruct((_NC, n_nodes, d), jnp.float32),
        mesh=mesh,
        scratch_types=[
            [pltpu.VMEM((_G, _CH), jnp.int32) for _ in range(2)],
            [pltpu.VMEM((_G, _CH), jnp.int32) for _ in range(2)],
            [pltpu.VMEM((_G, _CH), jnp.float32) for _ in range(2)],
            [pltpu.VMEM((_CH, d), jnp.float32) for _ in range(2)],
            pltpu.VMEM_SHARED((n_nodes, d), jnp.float32),
            [pltpu.SemaphoreType.DMA for _ in range(2)],
            [pltpu.SemaphoreType.DMA for _ in range(2)],
            [pltpu.SemaphoreType.DMA for _ in range(2)],
        ],
        compiler_params=pltpu.CompilerParams(
            use_tc_tiling_on_sc=(d % 128 == 0)),
    )
    def agg(feat_hbm, src_hbm, dst_hbm, ew_hbm, out_hbm,
            sblk, dblk, wblk, rows, acc, semi, gsem, ssem):
        cid = lax.axis_index("c")
        sid = lax.axis_index("s")
        wid = sid * _NC + cid
        base = wid * K

        def _fetch_idx(gg, gs):
            pltpu.async_copy(src_hbm.at[pl.ds(base + gg * _G, _G)],
                             sblk[gs], semi[gs])
            pltpu.async_copy(dst_hbm.at[pl.ds(base + gg * _G, _G)],
                             dblk[gs], semi[gs])
            pltpu.async_copy(ew_hbm.at[pl.ds(base + gg * _G, _G)],
                             wblk[gs], semi[gs])

        def _wait_idx(gs):
            pltpu.make_async_copy(src_hbm.at[pl.ds(base, _G)], sblk[gs],
                                  semi[gs]).wait()
            pltpu.make_async_copy(dst_hbm.at[pl.ds(base, _G)], dblk[gs],
                                  semi[gs]).wait()
            pltpu.make_async_copy(ew_hbm.at[pl.ds(base, _G)], wblk[gs],
                                  semi[gs]).wait()

        def _gather(gs, t, j):
            pltpu.async_copy(feat_hbm.at[sblk[gs].at[t]], rows[j], gsem[j])

        def _gather_wait(gs, j):
            pltpu.make_async_copy(feat_hbm.at[sblk[gs].at[0]], rows[j],
                                  gsem[j]).wait()

        def _scatter(gs, t, j):
            pltpu.async_copy(rows[j], acc.at[dblk[gs].at[t]], ssem[j],
                             add=True)

        def _scatter_wait(gs, j):
            pltpu.make_async_copy(rows[j], acc.at[dblk[gs].at[0]],
                                  ssem[j]).wait()

        def _scale(gs, t, j):
            def srow(m, _):
                w16 = wblk[gs][t, pl.ds(m * 16, 16)]
                for l in range(16):
                    w = w16[l]
                    i = m * 16 + l
                    for q in range(d // 16):
                        sl = pl.ds(q * 16, 16)
                        rows[j][i, sl] = rows[j][i, sl] * w
                return 0
            lax.fori_loop(0, _CH // 16, srow, 0)

        _fetch_idx(0, 0)
        if NG > 1:
            _fetch_idx(1, 1)

        def _zrow(i, _):
            for j in range(d // 16):
                rows[0][i, pl.ds(j * 16, 16)] = jnp.zeros((16,), jnp.float32)
            return 0
        lax.fori_loop(0, zc, _zrow, 0)

        def _zcopy(m, _):
            z = sid + m * _NS
            pltpu.sync_copy(rows[0].at[pl.ds(0, zc)], acc.at[pl.ds(z * zc, zc)])
            return 0
        lax.fori_loop(0, (n_zc - sid + _NS - 1) // _NS, _zcopy, 0)

        _wait_idx(0)
        _gather(0, 0, 0)
        _gather(0, 1, 1)
        plsc.subcore_barrier()

        for gg in range(NG):
            gs = gg % 2

            def inner(it, _):
                for j in range(2):
                    t = 2 * it + j
                    _gather_wait(gs, j)
                    _scale(gs, t, j)
                    _scatter(gs, t, j)
                for j in range(2):
                    t = 2 * it + j
                    _scatter_wait(gs, j)
                    _gather(gs, t + 2, j)
                return 0
            lax.fori_loop(0, (_G - 2) // 2, inner, 0)

            for j, t in ((0, _G - 2), (1, _G - 1)):
                _gather_wait(gs, j)
                _scale(gs, t, j)
                _scatter(gs, t, j)
            if gg < NG - 1:
                _wait_idx(1 - gs)
            for j in range(2):
                _scatter_wait(gs, j)
                if gg < NG - 1:
                    _gather(1 - gs, j, j)
            if gg < NG - 2:
                _fetch_idx(gg + 2, gs)

        plsc.subcore_barrier()

        def _ocopy(m, _):
            z = sid + m * _NS
            pltpu.sync_copy(acc.at[pl.ds(z * zc, zc)],
                            out_hbm.at[cid, pl.ds(z * zc, zc)])
            return 0
        lax.fori_loop(0, (n_zc - sid + _NS - 1) // _NS, _ocopy, 0)

    return agg


def _make_edge_agg_small(n_nodes: int, d: int, n_chunks: int):
    assert n_chunks % (_NW * _G) == 0
    K = n_chunks // _NW
    NG = K // _G
    zc = 80
    n_zc = n_nodes // zc
    assert n_nodes % zc == 0
    mesh = plsc.VectorSubcoreMesh(core_axis_name="c", subcore_axis_name="s")

    @functools.partial(
        pl.kernel,
        out_type=jax.ShapeDtypeStruct((_NC, n_nodes, d), jnp.float32),
        mesh=mesh,
        scratch_types=[
            [pltpu.VMEM((_G, _CH), jnp.int32) for _ in range(3)],
            [pltpu.VMEM((_G, _CH), jnp.int32) for _ in range(3)],
            [pltpu.VMEM((_G, _CH), jnp.float32) for _ in range(3)],
            [pltpu.VMEM((_G, _CH, d), jnp.float32) for _ in range(2)],
            pltpu.VMEM_SHARED((n_nodes, d), jnp.float32),
            [pltpu.SemaphoreType.DMA for _ in range(3)],
            [pltpu.SemaphoreType.DMA for _ in range(2)],
            [pltpu.SemaphoreType.DMA for _ in range(2)],
        ],
        compiler_params=pltpu.CompilerParams(use_tc_tiling_on_sc=False),
    )
    def agg(feat_hbm, src_hbm, dst_hbm, ew_hbm, out_hbm,
            sblk, dblk, wblk, rows, acc, semi, gsem, ssem):
        cid = lax.axis_index("c")
        sid = lax.axis_index("s")
        wid = sid * _NC + cid
        base = wid * K

        def _fetch_idx(gg, i3):
            pltpu.async_copy(src_hbm.at[pl.ds(base + gg * _G, _G)],
                             sblk[i3], semi[i3])
            pltpu.async_copy(dst_hbm.at[pl.ds(base + gg * _G, _G)],
                             dblk[i3], semi[i3])
            pltpu.async_copy(ew_hbm.at[pl.ds(base + gg * _G, _G)],
                             wblk[i3], semi[i3])

        def _wait_idx(i3):
            pltpu.make_async_copy(src_hbm.at[pl.ds(base, _G)], sblk[i3],
                                  semi[i3]).wait()
            pltpu.make_async_copy(dst_hbm.at[pl.ds(base, _G)], dblk[i3],
                                  semi[i3]).wait()
            pltpu.make_async_copy(ew_hbm.at[pl.ds(base, _G)], wblk[i3],
                                  semi[i3]).wait()

        def _gather(i3, j):
            for u in range(_G):
                pltpu.async_copy(feat_hbm.at[sblk[i3].at[u]], rows[j].at[u],
                                 gsem[j])

        def _gather_wait(i3, j):
            for u in range(_G):
                pltpu.make_async_copy(feat_hbm.at[sblk[i3].at[u]],
                                      rows[j].at[u], gsem[j]).wait()

        def _scatter(i3, j):
            for u in range(_G):
                pltpu.async_copy(rows[j].at[u], acc.at[dblk[i3].at[u]],
                                 ssem[j], add=True)

        def _scatter_wait(i3, j):
            for u in range(_G):
                pltpu.make_async_copy(rows[j].at[u], acc.at[dblk[i3].at[u]],
                                      ssem[j]).wait()

        def _scale(i3, j):
            def srow(t, _):
                u = t // (_CH // 16)
                m = t % (_CH // 16)
                w16 = wblk[i3][u, pl.ds(m * 16, 16)]
                for l in range(16):
                    w = w16[l]
                    i = m * 16 + l
                    sl = pl.ds(0, d)
                    rows[j][u, i, sl] = rows[j][u, i, sl] * w
                return 0
            lax.fori_loop(0, _G * (_CH // 16), srow, 0)

        _fetch_idx(0, 0)
        if NG > 1:
            _fetch_idx(1, 1)

        def _zrow(i, _):
            for q in range(d // 16):
                rows[0][0, i, pl.ds(q * 16, 16)] = jnp.zeros((16,), jnp.float32)
            return 0
        lax.fori_loop(0, zc, _zrow, 0)

        def _zcopy(m, _):
            z = sid + m * _NS
            pltpu.sync_copy(rows[0].at[0, pl.ds(0, zc)],
                            acc.at[pl.ds(z * zc, zc)])
            return 0
        lax.fori_loop(0, (n_zc - sid + _NS - 1) // _NS, _zcopy, 0)

        _wait_idx(0)
        _gather(0, 0)
        plsc.subcore_barrier()

        for g in range(NG):
            gs = g % 2
            i3 = g % 3
            _gather_wait(i3, gs)
            if g >= 1:
                _scatter_wait((g - 1) % 3, 1 - gs)
            if g + 1 < NG:
                _wait_idx((g + 1) % 3)
                _gather((g + 1) % 3, 1 - gs)
            if g + 2 < NG:
                _fetch_idx(g + 2, (g + 2) % 3)
            _scale(i3, gs)
            _scatter(i3, gs)

        _scatter_wait((NG - 1) % 3, (NG - 1) % 2)
        plsc.subcore_barrier()

        def _ocopy(m, _):
            z = sid + m * _NS
            pltpu.sync_copy(acc.at[pl.ds(z * zc, zc)],
                            out_hbm.at[cid, pl.ds(z * zc, zc)])
            return 0
        lax.fori_loop(0, (n_zc - sid + _NS - 1) // _NS, _ocopy, 0)

    return agg


def _linear2(P, W1, W2, n_nodes: int):
    R = 2000
    assert n_nodes % R == 0
    d = P.shape[2]
    n_cls = W2.shape[0]

    def mm(p_ref, w1_ref, w2_ref, g_ref):
        a = p_ref[0] + p_ref[1]
        h = lax.dot_general(a, w1_ref[...], (((1,), (1,)), ((), ())),
                            preferred_element_type=jnp.float32)
        h = jnp.maximum(h, 0.0)
        g_ref[...] = lax.dot_general(h, w2_ref[...], (((1,), (1,)), ((), ())),
                                     preferred_element_type=jnp.float32)

    return pl.pallas_call(
        mm,
        grid=(n_nodes // R,),
        in_specs=[
            pl.BlockSpec((2, R, d), lambda i: (0, i, 0)),
            pl.BlockSpec((d, d), lambda i: (0, 0)),
            pl.BlockSpec((n_cls, d), lambda i: (0, 0)),
        ],
        out_specs=pl.BlockSpec((R, n_cls), lambda i: (i, 0)),
        out_shape=jax.ShapeDtypeStruct((n_nodes, n_cls), jnp.float32),
    )(P, W1, W2)


def _log_softmax_sum(Q, n_nodes: int):
    R = 2000
    n_cls = Q.shape[2]

    def lsm(q_ref, o_ref):
        s = q_ref[0] + q_ref[1]
        m = jnp.max(s, axis=-1, keepdims=True)
        e = jnp.exp(s - m)
        lse = jnp.log(jnp.sum(e, axis=-1, keepdims=True))
        o_ref[...] = s - m - lse

    return pl.pallas_call(
        lsm,
        grid=(n_nodes // R,),
        in_specs=[pl.BlockSpec((2, R, n_cls), lambda i: (0, i, 0))],
        out_specs=pl.BlockSpec((R, n_cls), lambda i: (i, 0)),
        out_shape=jax.ShapeDtypeStruct((n_nodes, n_cls), jnp.float32),
    )(Q)


def kernel(x, edge_index, edge_weight, W1, W2):
    n_nodes, d_feat = x.shape
    n_edges = edge_weight.shape[0]
    grp = _CH * _NW * _G
    n_pad = -(-n_edges // grp) * grp - n_edges
    n_chunks = (n_edges + n_pad) // _CH

    pad_i = jnp.arange(n_pad, dtype=jnp.int32) % n_nodes
    src2d = jnp.concatenate(
        [edge_index[0].astype(jnp.int32), pad_i]).reshape(n_chunks, _CH)
    dst2d = jnp.concatenate(
        [edge_index[1].astype(jnp.int32), pad_i]).reshape(n_chunks, _CH)
    ew2d = jnp.concatenate(
        [edge_weight, jnp.zeros((n_pad,), jnp.float32)]).reshape(n_chunks, _CH)

    agg_x = _make_edge_agg(n_nodes, d_feat, n_chunks)
    P = agg_x(x, src2d, dst2d, ew2d)
    g = _linear2(P, W1, W2, n_nodes)
    agg_g = _make_edge_agg_small(n_nodes, W2.shape[0], n_chunks)
    Q = agg_g(g, src2d, dst2d, ew2d)
    return _log_softmax_sum(Q, n_nodes)


# speedup vs baseline: 14.0406x; 1.0275x over previous
import functools

import jax
import jax.numpy as jnp
from jax import lax
from jax.experimental import pallas as pl
from jax.experimental.pallas import tpu as pltpu
from jax.experimental.pallas import tpu_sc as plsc

_NC = 2
_NS = 16
_NW = _NC * _NS
_CH = 128


_G = 16


def _make_edge_agg(n_nodes: int, d: int, n_chunks: int):
    assert n_chunks % (_NW * _G) == 0
    K = n_chunks // _NW
    NG = K // _G
    zc = 80
    n_zc = n_nodes // zc
    assert n_nodes % zc == 0
    mesh = plsc.VectorSubcoreMesh(core_axis_name="c", subcore_axis_name="s")

    @functools.partial(
        pl.kernel,
        out_type=jax.ShapeDtypeStruct((_NC, n_nodes, d), jnp.float32),
        mesh=mesh,
        scratch_types=[
            [pltpu.VMEM((_G, _CH), jnp.int32) for _ in range(2)],
            [pltpu.VMEM((_G, _CH), jnp.int32) for _ in range(2)],
            [pltpu.VMEM((_G, _CH), jnp.float32) for _ in range(2)],
            [pltpu.VMEM((_CH, d), jnp.float32) for _ in range(2)],
            pltpu.VMEM_SHARED((n_nodes, d), jnp.float32),
            [pltpu.SemaphoreType.DMA for _ in range(2)],
            [pltpu.SemaphoreType.DMA for _ in range(2)],
            [pltpu.SemaphoreType.DMA for _ in range(2)],
        ],
        compiler_params=pltpu.CompilerParams(
            use_tc_tiling_on_sc=(d % 128 == 0)),
    )
    def agg(feat_hbm, ei_hbm, ew_hbm, out_hbm,
            sblk, dblk, wblk, rows, acc, semi, gsem, ssem):
        cid = lax.axis_index("c")
        sid = lax.axis_index("s")
        wid = sid * _NC + cid
        base = wid * K

        def _fetch_idx(gg, gs):
            pltpu.async_copy(ei_hbm.at[0, pl.ds(base + gg * _G, _G)],
                             sblk[gs], semi[gs])
            pltpu.async_copy(ei_hbm.at[1, pl.ds(base + gg * _G, _G)],
                             dblk[gs], semi[gs])
            pltpu.async_copy(ew_hbm.at[pl.ds(base + gg * _G, _G)],
                             wblk[gs], semi[gs])

        def _wait_idx(gs):
            pltpu.make_async_copy(ei_hbm.at[0, pl.ds(base, _G)], sblk[gs],
                                  semi[gs]).wait()
            pltpu.make_async_copy(ei_hbm.at[1, pl.ds(base, _G)], dblk[gs],
                                  semi[gs]).wait()
            pltpu.make_async_copy(ew_hbm.at[pl.ds(base, _G)], wblk[gs],
                                  semi[gs]).wait()

        def _gather(gs, t, j):
            pltpu.async_copy(feat_hbm.at[sblk[gs].at[t]], rows[j], gsem[j])

        def _gather_wait(gs, j):
            pltpu.make_async_copy(feat_hbm.at[sblk[gs].at[0]], rows[j],
                                  gsem[j]).wait()

        def _scatter(gs, t, j):
            pltpu.async_copy(rows[j], acc.at[dblk[gs].at[t]], ssem[j],
                             add=True)

        def _scatter_wait(gs, j):
            pltpu.make_async_copy(rows[j], acc.at[dblk[gs].at[0]],
                                  ssem[j]).wait()

        def _scale(gs, t, j):
            def srow(m, _):
                w16 = wblk[gs][t, pl.ds(m * 16, 16)]
                for l in range(16):
                    w = w16[l]
                    i = m * 16 + l
                    for q in range(d // 16):
                        sl = pl.ds(q * 16, 16)
                        rows[j][i, sl] = rows[j][i, sl] * w
                return 0
            lax.fori_loop(0, _CH // 16, srow, 0)

        _fetch_idx(0, 0)
        if NG > 1:
            _fetch_idx(1, 1)

        def _zrow(i, _):
            for j in range(d // 16):
                rows[0][i, pl.ds(j * 16, 16)] = jnp.zeros((16,), jnp.float32)
            return 0
        lax.fori_loop(0, zc, _zrow, 0)

        def _zcopy(m, _):
            z = sid + m * _NS
            pltpu.sync_copy(rows[0].at[pl.ds(0, zc)], acc.at[pl.ds(z * zc, zc)])
            return 0
        lax.fori_loop(0, (n_zc - sid + _NS - 1) // _NS, _zcopy, 0)

        _wait_idx(0)
        _gather(0, 0, 0)
        _gather(0, 1, 1)
        plsc.subcore_barrier()

        for gg in range(NG):
            gs = gg % 2

            def inner(it, _):
                for j in range(2):
                    t = 2 * it + j
                    _gather_wait(gs, j)
                    _scale(gs, t, j)
                    _scatter(gs, t, j)
                for j in range(2):
                    t = 2 * it + j
                    _scatter_wait(gs, j)
                    _gather(gs, t + 2, j)
                return 0
            lax.fori_loop(0, (_G - 2) // 2, inner, 0)

            for j, t in ((0, _G - 2), (1, _G - 1)):
                _gather_wait(gs, j)
                _scale(gs, t, j)
                _scatter(gs, t, j)
            if gg < NG - 1:
                _wait_idx(1 - gs)
            for j in range(2):
                _scatter_wait(gs, j)
                if gg < NG - 1:
                    _gather(1 - gs, j, j)
            if gg < NG - 2:
                _fetch_idx(gg + 2, gs)

        plsc.subcore_barrier()

        def _ocopy(m, _):
            z = sid + m * _NS
            pltpu.sync_copy(acc.at[pl.ds(z * zc, zc)],
                            out_hbm.at[cid, pl.ds(z * zc, zc)])
            return 0
        lax.fori_loop(0, (n_zc - sid + _NS - 1) // _NS, _ocopy, 0)

    return agg


def _make_edge_agg_small(n_nodes: int, d: int, n_chunks: int):
    assert n_chunks % (_NW * _G) == 0
    K = n_chunks // _NW
    NG = K // _G
    zc = 80
    n_zc = n_nodes // zc
    assert n_nodes % zc == 0
    mesh = plsc.VectorSubcoreMesh(core_axis_name="c", subcore_axis_name="s")

    @functools.partial(
        pl.kernel,
        out_type=jax.ShapeDtypeStruct((_NC, n_nodes, d), jnp.float32),
        mesh=mesh,
        scratch_types=[
            [pltpu.VMEM((_G, _CH), jnp.int32) for _ in range(3)],
            [pltpu.VMEM((_G, _CH), jnp.int32) for _ in range(3)],
            [pltpu.VMEM((_G, _CH), jnp.float32) for _ in range(3)],
            [pltpu.VMEM((_G, _CH, d), jnp.float32) for _ in range(2)],
            pltpu.VMEM_SHARED((n_nodes, d), jnp.float32),
            [pltpu.SemaphoreType.DMA for _ in range(3)],
            [pltpu.SemaphoreType.DMA for _ in range(2)],
            [pltpu.SemaphoreType.DMA for _ in range(2)],
        ],
        compiler_params=pltpu.CompilerParams(use_tc_tiling_on_sc=False),
    )
    def agg(feat_hbm, ei_hbm, ew_hbm, out_hbm,
            sblk, dblk, wblk, rows, acc, semi, gsem, ssem):
        cid = lax.axis_index("c")
        sid = lax.axis_index("s")
        wid = sid * _NC + cid
        base = wid * K

        def _fetch_idx(gg, i3):
            pltpu.async_copy(ei_hbm.at[0, pl.ds(base + gg * _G, _G)],
                             sblk[i3], semi[i3])
            pltpu.async_copy(ei_hbm.at[1, pl.ds(base + gg * _G, _G)],
                             dblk[i3], semi[i3])
            pltpu.async_copy(ew_hbm.at[pl.ds(base + gg * _G, _G)],
                             wblk[i3], semi[i3])

        def _wait_idx(i3):
            pltpu.make_async_copy(ei_hbm.at[0, pl.ds(base, _G)], sblk[i3],
                                  semi[i3]).wait()
            pltpu.make_async_copy(ei_hbm.at[1, pl.ds(base, _G)], dblk[i3],
                                  semi[i3]).wait()
            pltpu.make_async_copy(ew_hbm.at[pl.ds(base, _G)], wblk[i3],
                                  semi[i3]).wait()

        def _gather(i3, j):
            for u in range(_G):
                pltpu.async_copy(feat_hbm.at[sblk[i3].at[u]], rows[j].at[u],
                                 gsem[j])

        def _gather_wait(i3, j):
            for u in range(_G):
                pltpu.make_async_copy(feat_hbm.at[sblk[i3].at[u]],
                                      rows[j].at[u], gsem[j]).wait()

        def _scatter(i3, j):
            for u in range(_G):
                pltpu.async_copy(rows[j].at[u], acc.at[dblk[i3].at[u]],
                                 ssem[j], add=True)

        def _scatter_wait(i3, j):
            for u in range(_G):
                pltpu.make_async_copy(rows[j].at[u], acc.at[dblk[i3].at[u]],
                                      ssem[j]).wait()

        def _scale(i3, j):
            def srow(t, _):
                u = t // (_CH // 16)
                m = t % (_CH // 16)
                w16 = wblk[i3][u, pl.ds(m * 16, 16)]
                for l in range(16):
                    w = w16[l]
                    i = m * 16 + l
                    sl = pl.ds(0, d)
                    rows[j][u, i, sl] = rows[j][u, i, sl] * w
                return 0
            lax.fori_loop(0, _G * (_CH // 16), srow, 0)

        _fetch_idx(0, 0)
        if NG > 1:
            _fetch_idx(1, 1)

        def _zrow(i, _):
            for q in range(d // 16):
                rows[0][0, i, pl.ds(q * 16, 16)] = jnp.zeros((16,), jnp.float32)
            return 0
        lax.fori_loop(0, zc, _zrow, 0)

        def _zcopy(m, _):
            z = sid + m * _NS
            pltpu.sync_copy(rows[0].at[0, pl.ds(0, zc)],
                            acc.at[pl.ds(z * zc, zc)])
            return 0
        lax.fori_loop(0, (n_zc - sid + _NS - 1) // _NS, _zcopy, 0)

        _wait_idx(0)
        _gather(0, 0)
        plsc.subcore_barrier()

        for g in range(NG):
            gs = g % 2
            i3 = g % 3
            _gather_wait(i3, gs)
            if g >= 1:
                _scatter_wait((g - 1) % 3, 1 - gs)
            if g + 1 < NG:
                _wait_idx((g + 1) % 3)
                _gather((g + 1) % 3, 1 - gs)
            if g + 2 < NG:
                _fetch_idx(g + 2, (g + 2) % 3)
            _scale(i3, gs)
            _scatter(i3, gs)

        _scatter_wait((NG - 1) % 3, (NG - 1) % 2)
        plsc.subcore_barrier()

        def _ocopy(m, _):
            z = sid + m * _NS
            pltpu.sync_copy(acc.at[pl.ds(z * zc, zc)],
                            out_hbm.at[cid, pl.ds(z * zc, zc)])
            return 0
        lax.fori_loop(0, (n_zc - sid + _NS - 1) // _NS, _ocopy, 0)

    return agg


def _linear2(P, W1, W2, n_nodes: int):
    R = 2000
    assert n_nodes % R == 0
    d = P.shape[2]
    n_cls = W2.shape[0]

    def mm(p_ref, w1_ref, w2_ref, g_ref):
        a = p_ref[0] + p_ref[1]
        h = lax.dot_general(a, w1_ref[...], (((1,), (1,)), ((), ())),
                            preferred_element_type=jnp.float32)
        h = jnp.maximum(h, 0.0)
        g_ref[...] = lax.dot_general(h, w2_ref[...], (((1,), (1,)), ((), ())),
                                     preferred_element_type=jnp.float32)

    return pl.pallas_call(
        mm,
        grid=(n_nodes // R,),
        in_specs=[
            pl.BlockSpec((2, R, d), lambda i: (0, i, 0)),
            pl.BlockSpec((d, d), lambda i: (0, 0)),
            pl.BlockSpec((n_cls, d), lambda i: (0, 0)),
        ],
        out_specs=pl.BlockSpec((R, n_cls), lambda i: (i, 0)),
        out_shape=jax.ShapeDtypeStruct((n_nodes, n_cls), jnp.float32),
    )(P, W1, W2)


def _log_softmax_sum(Q, n_nodes: int):
    R = 2000
    n_cls = Q.shape[2]

    def lsm(q_ref, o_ref):
        s = q_ref[0] + q_ref[1]
        m = jnp.max(s, axis=-1, keepdims=True)
        e = jnp.exp(s - m)
        lse = jnp.log(jnp.sum(e, axis=-1, keepdims=True))
        o_ref[...] = s - m - lse

    return pl.pallas_call(
        lsm,
        grid=(n_nodes // R,),
        in_specs=[pl.BlockSpec((2, R, n_cls), lambda i: (0, i, 0))],
        out_specs=pl.BlockSpec((R, n_cls), lambda i: (i, 0)),
        out_shape=jax.ShapeDtypeStruct((n_nodes, n_cls), jnp.float32),
    )(Q)


def kernel(x, edge_index, edge_weight, W1, W2):
    n_nodes, d_feat = x.shape
    n_edges = edge_weight.shape[0]
    grp = _CH * _NW * _G
    n_pad = -(-n_edges // grp) * grp - n_edges
    n_chunks = (n_edges + n_pad) // _CH

    pad_i = jnp.arange(n_pad, dtype=jnp.int32) % n_nodes
    ei3d = jnp.concatenate(
        [edge_index.astype(jnp.int32),
         jnp.broadcast_to(pad_i, (2, n_pad))],
        axis=1).reshape(2, n_chunks, _CH)
    ew2d = jnp.concatenate(
        [edge_weight, jnp.zeros((n_pad,), jnp.float32)]).reshape(n_chunks, _CH)

    agg_x = _make_edge_agg(n_nodes, d_feat, n_chunks)
    P = agg_x(x, ei3d, ew2d)
    g = _linear2(P, W1, W2, n_nodes)
    agg_g = _make_edge_agg_small(n_nodes, W2.shape[0], n_chunks)
    Q = agg_g(g, ei3d, ew2d)
    return _log_softmax_sum(Q, n_nodes)


# speedup vs baseline: 14.1407x; 1.0071x over previous
import functools

import jax
import jax.numpy as jnp
from jax import lax
from jax.experimental import pallas as pl
from jax.experimental.pallas import tpu as pltpu
from jax.experimental.pallas import tpu_sc as plsc

_NC = 2
_NS = 16
_NW = _NC * _NS
_CH = 128


_G = 16


def _make_edge_agg(n_nodes: int, d: int, n_chunks: int):
    assert n_chunks % (_NW * _G) == 0
    K = n_chunks // _NW
    NG = K // _G
    zc = 80
    n_zc = n_nodes // zc
    assert n_nodes % zc == 0
    mesh = plsc.VectorSubcoreMesh(core_axis_name="c", subcore_axis_name="s")

    @functools.partial(
        pl.kernel,
        out_type=jax.ShapeDtypeStruct((_NC, n_nodes, d), jnp.float32),
        mesh=mesh,
        scratch_types=[
            [pltpu.VMEM((_G, _CH), jnp.int32) for _ in range(2)],
            [pltpu.VMEM((_G, _CH), jnp.int32) for _ in range(2)],
            [pltpu.VMEM((_G, _CH), jnp.float32) for _ in range(2)],
            [pltpu.VMEM((_CH, d), jnp.float32) for _ in range(2)],
            pltpu.VMEM_SHARED((n_nodes, d), jnp.float32),
            [pltpu.SemaphoreType.DMA for _ in range(2)],
            [pltpu.SemaphoreType.DMA for _ in range(2)],
            [pltpu.SemaphoreType.DMA for _ in range(2)],
        ],
        compiler_params=pltpu.CompilerParams(
            use_tc_tiling_on_sc=(d % 128 == 0)),
    )
    def agg(feat_hbm, ei_hbm, ew_hbm, out_hbm,
            sblk, dblk, wblk, rows, acc, semi, gsem, ssem):
        cid = lax.axis_index("c")
        sid = lax.axis_index("s")
        wid = sid * _NC + cid
        base = wid * K

        def _fetch_idx(gg, gs):
            pltpu.async_copy(ei_hbm.at[0, pl.ds(base + gg * _G, _G)],
                             sblk[gs], semi[gs])
            pltpu.async_copy(ei_hbm.at[1, pl.ds(base + gg * _G, _G)],
                             dblk[gs], semi[gs])
            pltpu.async_copy(ew_hbm.at[pl.ds(base + gg * _G, _G)],
                             wblk[gs], semi[gs])

        def _wait_idx(gs):
            pltpu.make_async_copy(ei_hbm.at[0, pl.ds(base, _G)], sblk[gs],
                                  semi[gs]).wait()
            pltpu.make_async_copy(ei_hbm.at[1, pl.ds(base, _G)], dblk[gs],
                                  semi[gs]).wait()
            pltpu.make_async_copy(ew_hbm.at[pl.ds(base, _G)], wblk[gs],
                                  semi[gs]).wait()

        def _gather(gs, t, j):
            pltpu.async_copy(feat_hbm.at[sblk[gs].at[t]], rows[j], gsem[j])

        def _gather_wait(gs, j):
            pltpu.make_async_copy(feat_hbm.at[sblk[gs].at[0]], rows[j],
                                  gsem[j]).wait()

        def _scatter(gs, t, j):
            pltpu.async_copy(rows[j], acc.at[dblk[gs].at[t]], ssem[j],
                             add=True)

        def _scatter_wait(gs, j):
            pltpu.make_async_copy(rows[j], acc.at[dblk[gs].at[0]],
                                  ssem[j]).wait()

        def _scale(gs, t, j):
            def srow(m, _):
                w16 = wblk[gs][t, pl.ds(m * 16, 16)]
                for l in range(16):
                    w = w16[l]
                    i = m * 16 + l
                    for q in range(d // 16):
                        sl = pl.ds(q * 16, 16)
                        rows[j][i, sl] = rows[j][i, sl] * w
                return 0
            lax.fori_loop(0, _CH // 16, srow, 0)

        _fetch_idx(0, 0)
        if NG > 1:
            _fetch_idx(1, 1)

        def _zrow(i, _):
            for j in range(d // 16):
                rows[0][i, pl.ds(j * 16, 16)] = jnp.zeros((16,), jnp.float32)
            return 0
        lax.fori_loop(0, zc, _zrow, 0)

        def _zcopy(m, _):
            z = sid + m * _NS
            pltpu.sync_copy(rows[0].at[pl.ds(0, zc)], acc.at[pl.ds(z * zc, zc)])
            return 0
        lax.fori_loop(0, (n_zc - sid + _NS - 1) // _NS, _zcopy, 0)

        _wait_idx(0)
        _gather(0, 0, 0)
        _gather(0, 1, 1)
        plsc.subcore_barrier()

        for gg in range(NG):
            gs = gg % 2

            def inner(it, _):
                for j in range(2):
                    t = 2 * it + j
                    _gather_wait(gs, j)
                    _scale(gs, t, j)
                    _scatter(gs, t, j)
                for j in range(2):
                    t = 2 * it + j
                    _scatter_wait(gs, j)
                    _gather(gs, t + 2, j)
                return 0
            lax.fori_loop(0, (_G - 2) // 2, inner, 0)

            for j, t in ((0, _G - 2), (1, _G - 1)):
                _gather_wait(gs, j)
                _scale(gs, t, j)
                _scatter(gs, t, j)
            if gg < NG - 1:
                _wait_idx(1 - gs)
            for j in range(2):
                _scatter_wait(gs, j)
                if gg < NG - 1:
                    _gather(1 - gs, j, j)
            if gg < NG - 2:
                _fetch_idx(gg + 2, gs)

        plsc.subcore_barrier()

        def _ocopy(m, _):
            z = sid + m * _NS
            pltpu.sync_copy(acc.at[pl.ds(z * zc, zc)],
                            out_hbm.at[cid, pl.ds(z * zc, zc)])
            return 0
        lax.fori_loop(0, (n_zc - sid + _NS - 1) // _NS, _ocopy, 0)

    return agg


def _make_edge_agg_small(n_nodes: int, d: int, n_chunks: int):
    assert n_chunks % (_NW * _G) == 0
    K = n_chunks // _NW
    NG = K // _G
    zc = 80
    n_zc = n_nodes // zc
    assert n_nodes % zc == 0
    mesh = plsc.VectorSubcoreMesh(core_axis_name="c", subcore_axis_name="s")

    @functools.partial(
        pl.kernel,
        out_type=jax.ShapeDtypeStruct((_NC, n_nodes, d), jnp.float32),
        mesh=mesh,
        scratch_types=[
            [pltpu.VMEM((_G, _CH), jnp.int32) for _ in range(3)],
            [pltpu.VMEM((_G, _CH), jnp.int32) for _ in range(3)],
            [pltpu.VMEM((_G, _CH), jnp.float32) for _ in range(3)],
            [pltpu.VMEM((_G, _CH, d), jnp.float32) for _ in range(2)],
            pltpu.VMEM_SHARED((n_nodes, d), jnp.float32),
            [pltpu.SemaphoreType.DMA for _ in range(3)],
            [pltpu.SemaphoreType.DMA for _ in range(2)],
            [pltpu.SemaphoreType.DMA for _ in range(2)],
        ],
        compiler_params=pltpu.CompilerParams(use_tc_tiling_on_sc=False),
    )
    def agg(feat_hbm, ei_hbm, ew_hbm, out_hbm,
            sblk, dblk, wblk, rows, acc, semi, gsem, ssem):
        cid = lax.axis_index("c")
        sid = lax.axis_index("s")
        wid = sid * _NC + cid
        base = wid * K

        def _fetch_idx(gg, i3):
            pltpu.async_copy(ei_hbm.at[0, pl.ds(base + gg * _G, _G)],
                             sblk[i3], semi[i3])
            pltpu.async_copy(ei_hbm.at[1, pl.ds(base + gg * _G, _G)],
                             dblk[i3], semi[i3])
            pltpu.async_copy(ew_hbm.at[pl.ds(base + gg * _G, _G)],
                             wblk[i3], semi[i3])

        def _wait_idx(i3):
            pltpu.make_async_copy(ei_hbm.at[0, pl.ds(base, _G)], sblk[i3],
                                  semi[i3]).wait()
            pltpu.make_async_copy(ei_hbm.at[1, pl.ds(base, _G)], dblk[i3],
                                  semi[i3]).wait()
            pltpu.make_async_copy(ew_hbm.at[pl.ds(base, _G)], wblk[i3],
                                  semi[i3]).wait()

        def _gather(i3, j):
            for u in range(_G):
                pltpu.async_copy(feat_hbm.at[sblk[i3].at[u]], rows[j].at[u],
                                 gsem[j])

        def _gather_wait(i3, j):
            for u in range(_G):
                pltpu.make_async_copy(feat_hbm.at[sblk[i3].at[u]],
                                      rows[j].at[u], gsem[j]).wait()

        def _scatter(i3, j):
            for u in range(_G):
                pltpu.async_copy(rows[j].at[u], acc.at[dblk[i3].at[u]],
                                 ssem[j], add=True)

        def _scatter_wait(i3, j):
            for u in range(_G):
                pltpu.make_async_copy(rows[j].at[u], acc.at[dblk[i3].at[u]],
                                      ssem[j]).wait()

        def _scale(i3, j):
            def srow(t, _):
                u = t // (_CH // 16)
                m = t % (_CH // 16)
                w16 = wblk[i3][u, pl.ds(m * 16, 16)]
                for l in range(16):
                    w = w16[l]
                    i = m * 16 + l
                    sl = pl.ds(0, d)
                    rows[j][u, i, sl] = rows[j][u, i, sl] * w
                return 0
            lax.fori_loop(0, _G * (_CH // 16), srow, 0)

        _fetch_idx(0, 0)
        if NG > 1:
            _fetch_idx(1, 1)

        def _zrow(i, _):
            for q in range(d // 16):
                rows[0][0, i, pl.ds(q * 16, 16)] = jnp.zeros((16,), jnp.float32)
            return 0
        lax.fori_loop(0, zc, _zrow, 0)

        def _zcopy(m, _):
            z = sid + m * _NS
            pltpu.sync_copy(rows[0].at[0, pl.ds(0, zc)],
                            acc.at[pl.ds(z * zc, zc)])
            return 0
        lax.fori_loop(0, (n_zc - sid + _NS - 1) // _NS, _zcopy, 0)

        _wait_idx(0)
        _gather(0, 0)
        plsc.subcore_barrier()

        for g in range(NG):
            gs = g % 2
            i3 = g % 3
            _gather_wait(i3, gs)
            if g >= 1:
                _scatter_wait((g - 1) % 3, 1 - gs)
            if g + 1 < NG:
                _wait_idx((g + 1) % 3)
                _gather((g + 1) % 3, 1 - gs)
            if g + 2 < NG:
                _fetch_idx(g + 2, (g + 2) % 3)
            _scale(i3, gs)
            _scatter(i3, gs)

        _scatter_wait((NG - 1) % 3, (NG - 1) % 2)
        plsc.subcore_barrier()

        def _ocopy(m, _):
            z = sid + m * _NS
            pltpu.sync_copy(acc.at[pl.ds(z * zc, zc)],
                            out_hbm.at[cid, pl.ds(z * zc, zc)])
            return 0
        lax.fori_loop(0, (n_zc - sid + _NS - 1) // _NS, _ocopy, 0)

    return agg


def _linear2(P, W1, W2, n_nodes: int):
    R = 1000
    assert n_nodes % R == 0
    d = P.shape[2]
    n_cls = W2.shape[0]

    def mm(p_ref, w1_ref, w2_ref, g_ref):
        a = p_ref[0] + p_ref[1]
        h = lax.dot_general(a, w1_ref[...], (((1,), (1,)), ((), ())),
                            preferred_element_type=jnp.float32)
        h = jnp.maximum(h, 0.0)
        g_ref[...] = lax.dot_general(h, w2_ref[...], (((1,), (1,)), ((), ())),
                                     preferred_element_type=jnp.float32)

    return pl.pallas_call(
        mm,
        grid=(n_nodes // R,),
        in_specs=[
            pl.BlockSpec((2, R, d), lambda i: (0, i, 0)),
            pl.BlockSpec((d, d), lambda i: (0, 0)),
            pl.BlockSpec((n_cls, d), lambda i: (0, 0)),
        ],
        out_specs=pl.BlockSpec((R, n_cls), lambda i: (i, 0)),
        out_shape=jax.ShapeDtypeStruct((n_nodes, n_cls), jnp.float32),
    )(P, W1, W2)


def _log_softmax_sum(Q, n_nodes: int):
    n_cls = Q.shape[2]
    pack = 128 // n_cls
    rows = n_nodes // pack
    Qw = Q.reshape(2, rows, 128)

    def lsm(q_ref, o_ref):
        s = q_ref[0] + q_ref[1]
        s3 = s.reshape(s.shape[0], pack, n_cls)
        m = jnp.max(s3, axis=-1, keepdims=True)
        e = jnp.exp(s3 - m)
        lse = jnp.log(jnp.sum(e, axis=-1, keepdims=True))
        o_ref[...] = (s3 - m - lse).reshape(s.shape[0], 128)

    out = pl.pallas_call(
        lsm,
        in_specs=[pl.BlockSpec((2, rows, 128), lambda: (0, 0, 0))],
        out_specs=pl.BlockSpec((rows, 128), lambda: (0, 0)),
        out_shape=jax.ShapeDtypeStruct((rows, 128), jnp.float32),
    )(Qw)
    return out.reshape(n_nodes, n_cls)


def kernel(x, edge_index, edge_weight, W1, W2):
    n_nodes, d_feat = x.shape
    n_edges = edge_weight.shape[0]
    grp = _CH * _NW * _G
    n_pad = -(-n_edges // grp) * grp - n_edges
    n_chunks = (n_edges + n_pad) // _CH

    pad_i = jnp.arange(n_pad, dtype=jnp.int32) % n_nodes
    ei3d = jnp.concatenate(
        [edge_index.astype(jnp.int32),
         jnp.broadcast_to(pad_i, (2, n_pad))],
        axis=1).reshape(2, n_chunks, _CH)
    ew2d = jnp.concatenate(
        [edge_weight, jnp.zeros((n_pad,), jnp.float32)]).reshape(n_chunks, _CH)

    agg_x = _make_edge_agg(n_nodes, d_feat, n_chunks)
    P = agg_x(x, ei3d, ew2d)
    g = _linear2(P, W1, W2, n_nodes)
    agg_g = _make_edge_agg_small(n_nodes, W2.shape[0], n_chunks)
    Q = agg_g(g, ei3d, ew2d)
    return _log_softmax_sum(Q, n_nodes)


# speedup vs baseline: 14.4732x; 1.0235x over previous
import functools

import jax
import jax.numpy as jnp
from jax import lax
from jax.experimental import pallas as pl
from jax.experimental.pallas import tpu as pltpu
from jax.experimental.pallas import tpu_sc as plsc

_NC = 2
_NS = 16
_NW = _NC * _NS
_CH = 128


_G = 16


def _make_edge_agg(n_nodes: int, d: int, n_chunks: int, n_idx_chunks: int):
    assert n_chunks % (_NW * _G) == 0
    K = n_chunks // _NW
    NG = K // _G
    iclamp = (n_idx_chunks - _G) // 8 * 8
    zc = 80
    n_zc = n_nodes // zc
    assert n_nodes % zc == 0
    mesh = plsc.VectorSubcoreMesh(core_axis_name="c", subcore_axis_name="s")

    @functools.partial(
        pl.kernel,
        out_type=jax.ShapeDtypeStruct((_NC, n_nodes, d), jnp.float32),
        mesh=mesh,
        scratch_types=[
            [pltpu.VMEM((_G, _CH), jnp.int32) for _ in range(2)],
            [pltpu.VMEM((_G, _CH), jnp.int32) for _ in range(2)],
            [pltpu.VMEM((_G, _CH), jnp.float32) for _ in range(2)],
            [pltpu.VMEM((_CH, d), jnp.float32) for _ in range(2)],
            pltpu.VMEM_SHARED((n_nodes, d), jnp.float32),
            [pltpu.SemaphoreType.DMA for _ in range(2)],
            [pltpu.SemaphoreType.DMA for _ in range(2)],
            [pltpu.SemaphoreType.DMA for _ in range(2)],
        ],
        compiler_params=pltpu.CompilerParams(
            use_tc_tiling_on_sc=(d % 128 == 0)),
    )
    def agg(feat_hbm, ei_hbm, ew_hbm, out_hbm,
            sblk, dblk, wblk, rows, acc, semi, gsem, ssem):
        cid = lax.axis_index("c")
        sid = lax.axis_index("s")
        wid = sid * _NC + cid
        base = wid * K

        def _fetch_idx(gg, gs):
            off = base + gg * _G
            off_i = pl.multiple_of(jnp.minimum(off, iclamp), 8)
            pltpu.async_copy(ei_hbm.at[0, pl.ds(off_i, _G)],
                             sblk[gs], semi[gs])
            pltpu.async_copy(ei_hbm.at[1, pl.ds(off_i, _G)],
                             dblk[gs], semi[gs])
            pltpu.async_copy(ew_hbm.at[pl.ds(off, _G)],
                             wblk[gs], semi[gs])

        def _wait_idx(gs):
            pltpu.make_async_copy(ei_hbm.at[0, pl.ds(base, _G)], sblk[gs],
                                  semi[gs]).wait()
            pltpu.make_async_copy(ei_hbm.at[1, pl.ds(base, _G)], dblk[gs],
                                  semi[gs]).wait()
            pltpu.make_async_copy(ew_hbm.at[pl.ds(base, _G)], wblk[gs],
                                  semi[gs]).wait()

        def _gather(gs, t, j):
            pltpu.async_copy(feat_hbm.at[sblk[gs].at[t]], rows[j], gsem[j])

        def _gather_wait(gs, j):
            pltpu.make_async_copy(feat_hbm.at[sblk[gs].at[0]], rows[j],
                                  gsem[j]).wait()

        def _scatter(gs, t, j):
            pltpu.async_copy(rows[j], acc.at[dblk[gs].at[t]], ssem[j],
                             add=True)

        def _scatter_wait(gs, j):
            pltpu.make_async_copy(rows[j], acc.at[dblk[gs].at[0]],
                                  ssem[j]).wait()

        def _scale(gs, t, j):
            def srow(m, _):
                w16 = wblk[gs][t, pl.ds(m * 16, 16)]
                for l in range(16):
                    w = w16[l]
                    i = m * 16 + l
                    for q in range(d // 16):
                        sl = pl.ds(q * 16, 16)
                        rows[j][i, sl] = rows[j][i, sl] * w
                return 0
            lax.fori_loop(0, _CH // 16, srow, 0)

        _fetch_idx(0, 0)
        if NG > 1:
            _fetch_idx(1, 1)

        def _zrow(i, _):
            for j in range(d // 16):
                rows[0][i, pl.ds(j * 16, 16)] = jnp.zeros((16,), jnp.float32)
            return 0
        lax.fori_loop(0, zc, _zrow, 0)

        def _zcopy(m, _):
            z = sid + m * _NS
            pltpu.sync_copy(rows[0].at[pl.ds(0, zc)], acc.at[pl.ds(z * zc, zc)])
            return 0
        lax.fori_loop(0, (n_zc - sid + _NS - 1) // _NS, _zcopy, 0)

        _wait_idx(0)
        _gather(0, 0, 0)
        _gather(0, 1, 1)
        plsc.subcore_barrier()

        for gg in range(NG):
            gs = gg % 2

            def inner(it, _):
                for j in range(2):
                    t = 2 * it + j
                    _gather_wait(gs, j)
                    _scale(gs, t, j)
                    _scatter(gs, t, j)
                for j in range(2):
                    t = 2 * it + j
                    _scatter_wait(gs, j)
                    _gather(gs, t + 2, j)
                return 0
            lax.fori_loop(0, (_G - 2) // 2, inner, 0)

            for j, t in ((0, _G - 2), (1, _G - 1)):
                _gather_wait(gs, j)
                _scale(gs, t, j)
                _scatter(gs, t, j)
            if gg < NG - 1:
                _wait_idx(1 - gs)
            for j in range(2):
                _scatter_wait(gs, j)
                if gg < NG - 1:
                    _gather(1 - gs, j, j)
            if gg < NG - 2:
                _fetch_idx(gg + 2, gs)

        plsc.subcore_barrier()

        def _ocopy(m, _):
            z = sid + m * _NS
            pltpu.sync_copy(acc.at[pl.ds(z * zc, zc)],
                            out_hbm.at[cid, pl.ds(z * zc, zc)])
            return 0
        lax.fori_loop(0, (n_zc - sid + _NS - 1) // _NS, _ocopy, 0)

    return agg


def _make_edge_agg_small(n_nodes: int, d: int, n_chunks: int, n_idx_chunks: int):
    assert n_chunks % (_NW * _G) == 0
    K = n_chunks // _NW
    NG = K // _G
    iclamp = (n_idx_chunks - _G) // 8 * 8
    zc = 80
    n_zc = n_nodes // zc
    assert n_nodes % zc == 0
    mesh = plsc.VectorSubcoreMesh(core_axis_name="c", subcore_axis_name="s")

    @functools.partial(
        pl.kernel,
        out_type=jax.ShapeDtypeStruct((_NC, n_nodes, d), jnp.float32),
        mesh=mesh,
        scratch_types=[
            [pltpu.VMEM((_G, _CH), jnp.int32) for _ in range(3)],
            [pltpu.VMEM((_G, _CH), jnp.int32) for _ in range(3)],
            [pltpu.VMEM((_G, _CH), jnp.float32) for _ in range(3)],
            [pltpu.VMEM((_G, _CH, d), jnp.float32) for _ in range(2)],
            pltpu.VMEM_SHARED((n_nodes, d), jnp.float32),
            [pltpu.SemaphoreType.DMA for _ in range(3)],
            [pltpu.SemaphoreType.DMA for _ in range(2)],
            [pltpu.SemaphoreType.DMA for _ in range(2)],
        ],
        compiler_params=pltpu.CompilerParams(use_tc_tiling_on_sc=False),
    )
    def agg(feat_hbm, ei_hbm, ew_hbm, out_hbm,
            sblk, dblk, wblk, rows, acc, semi, gsem, ssem):
        cid = lax.axis_index("c")
        sid = lax.axis_index("s")
        wid = sid * _NC + cid
        base = wid * K

        def _fetch_idx(gg, i3):
            off = base + gg * _G
            off_i = pl.multiple_of(jnp.minimum(off, iclamp), 8)
            pltpu.async_copy(ei_hbm.at[0, pl.ds(off_i, _G)],
                             sblk[i3], semi[i3])
            pltpu.async_copy(ei_hbm.at[1, pl.ds(off_i, _G)],
                             dblk[i3], semi[i3])
            pltpu.async_copy(ew_hbm.at[pl.ds(off, _G)],
                             wblk[i3], semi[i3])

        def _wait_idx(i3):
            pltpu.make_async_copy(ei_hbm.at[0, pl.ds(base, _G)], sblk[i3],
                                  semi[i3]).wait()
            pltpu.make_async_copy(ei_hbm.at[1, pl.ds(base, _G)], dblk[i3],
                                  semi[i3]).wait()
            pltpu.make_async_copy(ew_hbm.at[pl.ds(base, _G)], wblk[i3],
                                  semi[i3]).wait()

        def _gather(i3, j):
            for u in range(_G):
                pltpu.async_copy(feat_hbm.at[sblk[i3].at[u]], rows[j].at[u],
                                 gsem[j])

        def _gather_wait(i3, j):
            for u in range(_G):
                pltpu.make_async_copy(feat_hbm.at[sblk[i3].at[u]],
                                      rows[j].at[u], gsem[j]).wait()

        def _scatter(i3, j):
            for u in range(_G):
                pltpu.async_copy(rows[j].at[u], acc.at[dblk[i3].at[u]],
                                 ssem[j], add=True)

        def _scatter_wait(i3, j):
            for u in range(_G):
                pltpu.make_async_copy(rows[j].at[u], acc.at[dblk[i3].at[u]],
                                      ssem[j]).wait()

        def _scale(i3, j):
            def srow(t, _):
                u = t // (_CH // 16)
                m = t % (_CH // 16)
                w16 = wblk[i3][u, pl.ds(m * 16, 16)]
                for l in range(16):
                    w = w16[l]
                    i = m * 16 + l
                    sl = pl.ds(0, d)
                    rows[j][u, i, sl] = rows[j][u, i, sl] * w
                return 0
            lax.fori_loop(0, _G * (_CH // 16), srow, 0)

        _fetch_idx(0, 0)
        if NG > 1:
            _fetch_idx(1, 1)

        def _zrow(i, _):
            for q in range(d // 16):
                rows[0][0, i, pl.ds(q * 16, 16)] = jnp.zeros((16,), jnp.float32)
            return 0
        lax.fori_loop(0, zc, _zrow, 0)

        def _zcopy(m, _):
            z = sid + m * _NS
            pltpu.sync_copy(rows[0].at[0, pl.ds(0, zc)],
                            acc.at[pl.ds(z * zc, zc)])
            return 0
        lax.fori_loop(0, (n_zc - sid + _NS - 1) // _NS, _zcopy, 0)

        _wait_idx(0)
        _gather(0, 0)
        plsc.subcore_barrier()

        for g in range(NG):
            gs = g % 2
            i3 = g % 3
            _gather_wait(i3, gs)
            if g >= 1:
                _scatter_wait((g - 1) % 3, 1 - gs)
            if g + 1 < NG:
                _wait_idx((g + 1) % 3)
                _gather((g + 1) % 3, 1 - gs)
            if g + 2 < NG:
                _fetch_idx(g + 2, (g + 2) % 3)
            _scale(i3, gs)
            _scatter(i3, gs)

        _scatter_wait((NG - 1) % 3, (NG - 1) % 2)
        plsc.subcore_barrier()

        def _ocopy(m, _):
            z = sid + m * _NS
            pltpu.sync_copy(acc.at[pl.ds(z * zc, zc)],
                            out_hbm.at[cid, pl.ds(z * zc, zc)])
            return 0
        lax.fori_loop(0, (n_zc - sid + _NS - 1) // _NS, _ocopy, 0)

    return agg


def _linear2(P, W1, W2, n_nodes: int):
    R = 2000
    assert n_nodes % R == 0
    d = P.shape[2]
    n_cls = W2.shape[0]

    def mm(p_ref, w1_ref, w2_ref, g_ref):
        a = p_ref[0] + p_ref[1]
        h = lax.dot_general(a, w1_ref[...], (((1,), (1,)), ((), ())),
                            preferred_element_type=jnp.float32)
        h = jnp.maximum(h, 0.0)
        g_ref[...] = lax.dot_general(h, w2_ref[...], (((1,), (1,)), ((), ())),
                                     preferred_element_type=jnp.float32)

    return pl.pallas_call(
        mm,
        grid=(n_nodes // R,),
        in_specs=[
            pl.BlockSpec((2, R, d), lambda i: (0, i, 0)),
            pl.BlockSpec((d, d), lambda i: (0, 0)),
            pl.BlockSpec((n_cls, d), lambda i: (0, 0)),
        ],
        out_specs=pl.BlockSpec((R, n_cls), lambda i: (i, 0)),
        out_shape=jax.ShapeDtypeStruct((n_nodes, n_cls), jnp.float32),
    )(P, W1, W2)


def _log_softmax_sum(Q, n_nodes: int):
    n_cls = Q.shape[2]
    pack = 128 // n_cls
    rows = n_nodes // pack
    Qw = Q.reshape(2, rows, 128)

    def lsm(q_ref, o_ref):
        s = q_ref[0] + q_ref[1]
        s3 = s.reshape(s.shape[0], pack, n_cls)
        m = jnp.max(s3, axis=-1, keepdims=True)
        e = jnp.exp(s3 - m)
        lse = jnp.log(jnp.sum(e, axis=-1, keepdims=True))
        o_ref[...] = (s3 - m - lse).reshape(s.shape[0], 128)

    out = pl.pallas_call(
        lsm,
        in_specs=[pl.BlockSpec((2, rows, 128), lambda: (0, 0, 0))],
        out_specs=pl.BlockSpec((rows, 128), lambda: (0, 0)),
        out_shape=jax.ShapeDtypeStruct((rows, 128), jnp.float32),
    )(Qw)
    return out.reshape(n_nodes, n_cls)


def kernel(x, edge_index, edge_weight, W1, W2):
    n_nodes, d_feat = x.shape
    n_edges = edge_weight.shape[0]
    grp = _CH * _NW * _G
    n_pad = -(-n_edges // grp) * grp - n_edges
    n_chunks = (n_edges + n_pad) // _CH

    assert n_edges % _CH == 0
    n_idx_chunks = n_edges // _CH
    ei3d = edge_index.astype(jnp.int32).reshape(2, n_idx_chunks, _CH)
    ew2d = jnp.concatenate(
        [edge_weight, jnp.zeros((n_pad,), jnp.float32)]).reshape(n_chunks, _CH)

    agg_x = _make_edge_agg(n_nodes, d_feat, n_chunks, n_idx_chunks)
    P = agg_x(x, ei3d, ew2d)
    g = _linear2(P, W1, W2, n_nodes)
    agg_g = _make_edge_agg_small(n_nodes, W2.shape[0], n_chunks, n_idx_chunks)
    Q = agg_g(g, ei3d, ew2d)
    return _log_softmax_sum(Q, n_nodes)
